# Initial kernel scaffold; baseline (speedup 1.0000x reference)
import jax
import jax.numpy as jnp
from jax import lax
from jax.experimental import pallas as pl
from jax.experimental.pallas import tpu as pltpu
from jax.experimental.pallas import tpu_sc as plsc

NC, NS = 2, 16
NW = NC * NS
CHUNK = 128
BATCH, SEQ, D = 4096, 26, 64
B = BATCH * SEQ
CPW = B // (NW * CHUNK)

_mesh = plsc.VectorSubcoreMesh(
    core_axis_name="c", subcore_axis_name="s", num_cores=NC, num_subcores=NS
)


def _gather_body(ids_hbm, table_hbm, out_hbm, idx_v, rows_v, gsem):
    wid = lax.axis_index("s") * NC + lax.axis_index("c")
    pltpu.sync_copy(ids_hbm.at[pl.ds(wid * CPW, CPW)], idx_v)
    base = wid * (CPW * CHUNK)

    def step(j, carry):
        pltpu.async_copy(table_hbm.at[idx_v.at[j]], rows_v, gsem).wait()
        pltpu.sync_copy(rows_v, out_hbm.at[pl.ds(base + j * CHUNK, CHUNK)])
        return carry

    lax.fori_loop(0, CPW, step, 0)


_gather = pl.kernel(
    _gather_body,
    out_type=jax.ShapeDtypeStruct((B, D), jnp.float32),
    mesh=_mesh,
    scratch_types=[
        pltpu.VMEM((CPW, CHUNK), jnp.int32),
        pltpu.VMEM((CHUNK, D), jnp.float32),
        pltpu.SemaphoreType.DMA,
    ],
)


@jax.jit
def kernel(input_ids, table):
    ids = input_ids.astype(jnp.int32).reshape(NW * CPW, CHUNK)
    out = _gather(ids, table)
    return out.reshape(BATCH, SEQ, D)


# speedup vs baseline: 1.1035x; 1.1035x over previous
import jax
import jax.numpy as jnp
from jax import lax
from jax.experimental import pallas as pl
from jax.experimental.pallas import tpu as pltpu
from jax.experimental.pallas import tpu_sc as plsc

NC, NS = 2, 16
NW = NC * NS
CHUNK = 128
BATCH, SEQ, D = 4096, 26, 64
B = BATCH * SEQ
CPW = B // (NW * CHUNK)

_mesh = plsc.VectorSubcoreMesh(
    core_axis_name="c", subcore_axis_name="s", num_cores=NC, num_subcores=NS
)


BPW = CPW * CHUNK


def _gather_body(ids_hbm, table_hbm, out_hbm, idx_v, rows_v, gsem):
    wid = lax.axis_index("s") * NC + lax.axis_index("c")
    base = wid * BPW
    pltpu.sync_copy(ids_hbm.at[pl.ds(base, BPW)], idx_v)

    def step(j, carry):
        idx = idx_v.at[pl.ds(j * CHUNK, CHUNK)]
        pltpu.async_copy(table_hbm.at[idx], rows_v, gsem).wait()
        pltpu.sync_copy(rows_v, out_hbm.at[pl.ds(base + j * CHUNK, CHUNK)])
        return carry

    lax.fori_loop(0, CPW, step, 0)


_gather = pl.kernel(
    _gather_body,
    out_type=jax.ShapeDtypeStruct((B, D), jnp.float32),
    mesh=_mesh,
    scratch_types=[
        pltpu.VMEM((BPW,), jnp.int32),
        pltpu.VMEM((CHUNK, D), jnp.float32),
        pltpu.SemaphoreType.DMA,
    ],
    compiler_params=pltpu.CompilerParams(use_tc_tiling_on_sc=False),
)


@jax.jit
def kernel(input_ids, table):
    ids = input_ids.astype(jnp.int32).reshape(B)
    out = _gather(ids, table)
    return out.reshape(BATCH, SEQ, D)


# speedup vs baseline: 1.2120x; 1.0983x over previous
import jax
import jax.numpy as jnp
from jax import lax
from jax.experimental import pallas as pl
from jax.experimental.pallas import tpu as pltpu
from jax.experimental.pallas import tpu_sc as plsc

NC, NS = 2, 16
NW = NC * NS
BATCH, SEQ, D = 4096, 26, 64
B = BATCH * SEQ
BPW = B // NW
CHUNK = 104
CPW = BPW // CHUNK
GSZ = 4
NG = CPW // GSZ

_mesh = plsc.VectorSubcoreMesh(
    core_axis_name="c", subcore_axis_name="s", num_cores=NC, num_subcores=NS
)


def _gather_body(ids_hbm, table_hbm, out_hbm, idx_v, rows_v, gsem0, gsem1):
    wid = lax.axis_index("s") * NC + lax.axis_index("c")
    base = wid * BPW
    pltpu.sync_copy(ids_hbm.at[pl.ds(base, BPW)], idx_v)

    def fire(g, bank, sem):
        for s in range(GSZ):
            off = (g * GSZ + s) * CHUNK
            idx = idx_v.at[pl.ds(off, CHUNK)]
            pltpu.async_copy(table_hbm.at[idx], rows_v.at[bank, s], sem)

    def drain_store(g, bank, sem):
        for s in range(GSZ):
            off = (g * GSZ + s) * CHUNK
            idx = idx_v.at[pl.ds(off, CHUNK)]
            pltpu.make_async_copy(table_hbm.at[idx], rows_v.at[bank, s], sem).wait()
            pltpu.sync_copy(rows_v.at[bank, s], out_hbm.at[pl.ds(base + off, CHUNK)])

    fire(0, 0, gsem0)

    def body(h, carry):
        g0 = 2 * h
        fire(g0 + 1, 1, gsem1)
        drain_store(g0, 0, gsem0)

        @pl.when(h + 1 < NG // 2)
        def _():
            fire(g0 + 2, 0, gsem0)

        drain_store(g0 + 1, 1, gsem1)
        return carry

    lax.fori_loop(0, NG // 2, body, 0)


_gather = pl.kernel(
    _gather_body,
    out_type=jax.ShapeDtypeStruct((B, D), jnp.float32),
    mesh=_mesh,
    scratch_types=[
        pltpu.VMEM((BPW,), jnp.int32),
        pltpu.VMEM((2, GSZ, CHUNK, D), jnp.float32),
        pltpu.SemaphoreType.DMA,
        pltpu.SemaphoreType.DMA,
    ],
    compiler_params=pltpu.CompilerParams(use_tc_tiling_on_sc=False),
)


@jax.jit
def kernel(input_ids, table):
    ids = input_ids.astype(jnp.int32).reshape(B)
    out = _gather(ids, table)
    return out.reshape(BATCH, SEQ, D)
